# baseline (speedup 1.0000x reference)
import functools

import jax
import jax.numpy as jnp
from jax import lax
from jax.experimental import pallas as pl
from jax.experimental.pallas import tpu as pltpu
from jax.experimental.pallas import tpu_sc as plsc


def _sc_gather3(state_emb, act_emb_self, act_emb_other, sidx, aidx, oidx,
                H, h_off):
    D = state_emb.shape[1]
    info = plsc.get_sparse_core_info()
    NC, NS = info.num_cores, info.num_subcores
    NW = NC * NS
    assert H % (16 * NW) == 0
    b_per_w = H // NW
    ch = b_per_w // 2

    mesh = plsc.VectorSubcoreMesh(core_axis_name="c", subcore_axis_name="s")
    row_t = jax.ShapeDtypeStruct((H, D), jnp.float32)

    @functools.partial(
        pl.kernel,
        mesh=mesh,
        out_type=[row_t, row_t, row_t],
        scratch_types=[
            pltpu.VMEM((3 * b_per_w,), jnp.int32),
            pltpu.VMEM((ch, D), jnp.float32),
            pltpu.VMEM((ch, D), jnp.float32),
            pltpu.VMEM((ch, D), jnp.float32),
            pltpu.SemaphoreType.DMA,
            pltpu.SemaphoreType.DMA,
            pltpu.SemaphoreType.DMA,
            pltpu.SemaphoreType.DMA,
            pltpu.SemaphoreType.DMA,
            pltpu.SemaphoreType.DMA,
            pltpu.SemaphoreType.DMA,
        ],
    )
    def gather_k(state_hbm, aself_hbm, aother_hbm, sidx_hbm, aidx_hbm,
                 oidx_hbm, out_s, out_a, out_o, idx_v, buf0, buf1, buf2,
                 g0, g1, g2, w0, w1, w2, isem):
        wid = lax.axis_index("s") * NC + lax.axis_index("c")
        lbase = wid * b_per_w
        gbase = h_off + lbase

        tables = [state_hbm, aself_hbm, aother_hbm]
        idxs = [sidx_hbm, aidx_hbm, oidx_hbm]
        outs = [out_s, out_a, out_o]
        bufs = [buf0, buf1, buf2]
        gsems = [g0, g1, g2]
        wsems = [w0, w1, w2]

        icopies = [
            pltpu.make_async_copy(
                idxs[t].at[pl.ds(gbase, b_per_w)],
                idx_v.at[pl.ds(t * b_per_w, b_per_w)], isem)
            for t in range(3)
        ]
        for c in icopies:
            c.start()
        for c in icopies:
            c.wait()

        gathers = [None] * 6
        writes = [None] * 6
        for c in range(3):
            t, hh = c // 2, c % 2
            gathers[c] = pltpu.make_async_copy(
                tables[t].at[idx_v.at[pl.ds(t * b_per_w + hh * ch, ch)]],
                bufs[c % 3], gsems[c % 3])
            gathers[c].start()

        for c in range(6):
            t, hh = c // 2, c % 2
            gathers[c].wait()
            writes[c] = pltpu.make_async_copy(
                bufs[c % 3],
                outs[t].at[pl.ds(lbase + hh * ch, ch)], wsems[c % 3])
            writes[c].start()
            nc = c + 3
            if nc < 6:
                tn, hn = nc // 2, nc % 2
                writes[c].wait()
                gathers[nc] = pltpu.make_async_copy(
                    tables[tn].at[idx_v.at[pl.ds(tn * b_per_w + hn * ch, ch)]],
                    bufs[nc % 3], gsems[nc % 3])
                gathers[nc].start()

        for c in range(3, 6):
            writes[c].wait()

    return gather_k(state_emb, act_emb_self, act_emb_other, sidx, aidx, oidx)


def _tc_mlp(S, A, O, W1, b1, W2, b2, block_m):
    H, D = S.shape
    n_blocks = H // block_m

    def body(s_ref, a_ref, o_ref, w1_ref, b1_ref, w2_ref, b2_ref, out_ref):
        h = jnp.dot(s_ref[...], w1_ref[0:D, :],
                    preferred_element_type=jnp.float32)
        h = h + jnp.dot(a_ref[...], w1_ref[D:2 * D, :],
                        preferred_element_type=jnp.float32)
        h = h + jnp.dot(o_ref[...], w1_ref[2 * D:3 * D, :],
                        preferred_element_type=jnp.float32)
        h = jnp.maximum(h + b1_ref[...], 0.0)
        r = lax.dot_general(w2_ref[...], h, (((0,), (1,)), ((), ())),
                            preferred_element_type=jnp.float32)
        out_ref[...] = r + b2_ref[0]

    out = pl.pallas_call(
        body,
        grid=(n_blocks,),
        in_specs=[
            pl.BlockSpec((block_m, D), lambda i: (i, 0)),
            pl.BlockSpec((block_m, D), lambda i: (i, 0)),
            pl.BlockSpec((block_m, D), lambda i: (i, 0)),
            pl.BlockSpec((3 * D, D), lambda i: (0, 0)),
            pl.BlockSpec((1, D), lambda i: (0, 0)),
            pl.BlockSpec((D, 1), lambda i: (0, 0)),
            pl.BlockSpec(memory_space=pltpu.SMEM),
        ],
        out_specs=pl.BlockSpec((1, block_m), lambda i: (0, i)),
        out_shape=jax.ShapeDtypeStruct((1, H), jnp.float32),
        compiler_params=pltpu.CompilerParams(
            dimension_semantics=("arbitrary",),
        ),
    )(S, A, O, W1, b1, W2, b2)
    return out.reshape(H)


def kernel(state_indices, joint_actions, state_emb, act_emb_self,
           act_emb_other, W1, b1, W2, b2):
    B = state_indices.shape[0]
    H = B // 2
    sidx = state_indices.astype(jnp.int32)
    aidx = joint_actions[:, 0].astype(jnp.int32)
    oidx = joint_actions[:, 1].astype(jnp.int32)
    b1r = b1.reshape(1, -1)

    halves = []
    for h_off in (0, H):
        S, A, O = _sc_gather3(state_emb, act_emb_self, act_emb_other,
                              sidx, aidx, oidx, H, h_off)
        halves.append((S, A, O))

    outs = [_tc_mlp(S, A, O, W1, b1r, W2, b2, block_m=2048)
            for (S, A, O) in halves]
    return jnp.concatenate(outs)


# speedup vs baseline: 1.2952x; 1.2952x over previous
import functools

import jax
import jax.numpy as jnp
from jax import lax
from jax.experimental import pallas as pl
from jax.experimental.pallas import tpu as pltpu
from jax.experimental.pallas import tpu_sc as plsc


def _sc_gather3(state_emb, act_emb_self, act_emb_other, sidx, aidx, oidx):
    B = sidx.shape[0]
    D = state_emb.shape[1]
    NA = act_emb_self.shape[0]
    info = plsc.get_sparse_core_info()
    NC, NS = info.num_cores, info.num_subcores
    NW = NC * NS
    assert B % (16 * NW) == 0
    b_per_w = B // NW
    ch = b_per_w // 2

    mesh = plsc.VectorSubcoreMesh(core_axis_name="c", subcore_axis_name="s")
    row_t = jax.ShapeDtypeStruct((B, D), jnp.float32)

    @functools.partial(
        pl.kernel,
        mesh=mesh,
        out_type=[row_t, row_t, row_t],
        scratch_types=[
            pltpu.VMEM((3 * b_per_w,), jnp.int32),
            pltpu.VMEM((ch, D), jnp.float32),
            pltpu.VMEM((ch, D), jnp.float32),
            pltpu.VMEM((ch, D), jnp.float32),
            pltpu.VMEM_SHARED((NA, D), jnp.float32),
            pltpu.VMEM_SHARED((NA, D), jnp.float32),
            pltpu.SemaphoreType.DMA,
            pltpu.SemaphoreType.DMA,
            pltpu.SemaphoreType.DMA,
            pltpu.SemaphoreType.DMA,
            pltpu.SemaphoreType.DMA,
            pltpu.SemaphoreType.DMA,
            pltpu.SemaphoreType.DMA,
            pltpu.SemaphoreType.DMA,
        ],
    )
    def gather_k(state_hbm, aself_hbm, aother_hbm, sidx_hbm, aidx_hbm,
                 oidx_hbm, out_s, out_a, out_o, idx_v, buf0, buf1, buf2,
                 sh_a, sh_o, g0, g1, g2, w0, w1, w2, isem, tsem):
        wid = lax.axis_index("s") * NC + lax.axis_index("c")
        base = wid * b_per_w

        idxs = [sidx_hbm, aidx_hbm, oidx_hbm]
        outs = [out_s, out_a, out_o]
        bufs = [buf0, buf1, buf2]
        gsems = [g0, g1, g2]
        wsems = [w0, w1, w2]

        sid = lax.axis_index("s")
        tcopies = [
            pltpu.make_async_copy(aself_hbm, sh_a, tsem),
            pltpu.make_async_copy(aother_hbm, sh_o, tsem),
        ]

        @pl.when(sid == 0)
        def _():
            for t in tcopies:
                t.start()

        icopies = [
            pltpu.make_async_copy(
                idxs[t].at[pl.ds(base, b_per_w)],
                idx_v.at[pl.ds(t * b_per_w, b_per_w)], isem)
            for t in range(3)
        ]
        for c in icopies:
            c.start()
        for c in icopies:
            c.wait()

        def make_gather(c):
            t, hh = c // 2, c % 2
            tbl = [state_hbm, sh_a, sh_o][t]
            return pltpu.make_async_copy(
                tbl.at[idx_v.at[pl.ds(t * b_per_w + hh * ch, ch)]],
                bufs[c % 3], gsems[c % 3])

        gathers = [None] * 6
        writes = [None] * 6
        for c in range(2):
            gathers[c] = make_gather(c)
            gathers[c].start()

        @pl.when(sid == 0)
        def _():
            for t in tcopies:
                t.wait()
        plsc.subcore_barrier()

        gathers[2] = make_gather(2)
        gathers[2].start()

        for c in range(6):
            t, hh = c // 2, c % 2
            gathers[c].wait()
            writes[c] = pltpu.make_async_copy(
                bufs[c % 3],
                outs[t].at[pl.ds(base + hh * ch, ch)], wsems[c % 3])
            writes[c].start()
            nc = c + 3
            if nc < 6:
                writes[c].wait()
                gathers[nc] = make_gather(nc)
                gathers[nc].start()

        for c in range(3, 6):
            writes[c].wait()

    return gather_k(state_emb, act_emb_self, act_emb_other, sidx, aidx, oidx)


def _tc_mlp(S, A, O, W1, b1, W2, b2, block_m):
    B, D = S.shape
    n_blocks = B // block_m

    def body(s_ref, a_ref, o_ref, w1_ref, b1_ref, w2_ref, b2_ref, out_ref):
        h = jnp.dot(s_ref[...], w1_ref[0:D, :],
                    preferred_element_type=jnp.float32)
        h = h + jnp.dot(a_ref[...], w1_ref[D:2 * D, :],
                        preferred_element_type=jnp.float32)
        h = h + jnp.dot(o_ref[...], w1_ref[2 * D:3 * D, :],
                        preferred_element_type=jnp.float32)
        h = jnp.maximum(h + b1_ref[...], 0.0)
        r = lax.dot_general(w2_ref[...], h, (((0,), (1,)), ((), ())),
                            preferred_element_type=jnp.float32)
        out_ref[...] = r + b2_ref[0]

    out = pl.pallas_call(
        body,
        grid=(n_blocks,),
        in_specs=[
            pl.BlockSpec((block_m, D), lambda i: (i, 0)),
            pl.BlockSpec((block_m, D), lambda i: (i, 0)),
            pl.BlockSpec((block_m, D), lambda i: (i, 0)),
            pl.BlockSpec((3 * D, D), lambda i: (0, 0)),
            pl.BlockSpec((1, D), lambda i: (0, 0)),
            pl.BlockSpec((D, 1), lambda i: (0, 0)),
            pl.BlockSpec(memory_space=pltpu.SMEM),
        ],
        out_specs=pl.BlockSpec((1, block_m), lambda i: (0, i)),
        out_shape=jax.ShapeDtypeStruct((1, B), jnp.float32),
        compiler_params=pltpu.CompilerParams(
            dimension_semantics=("arbitrary",),
        ),
    )(S, A, O, W1, b1, W2, b2)
    return out.reshape(B)


def kernel(state_indices, joint_actions, state_emb, act_emb_self,
           act_emb_other, W1, b1, W2, b2):
    sidx = state_indices.astype(jnp.int32)
    aidx = joint_actions[:, 0].astype(jnp.int32)
    oidx = joint_actions[:, 1].astype(jnp.int32)

    S, A, O = _sc_gather3(state_emb, act_emb_self, act_emb_other,
                          sidx, aidx, oidx)

    b1r = b1.reshape(1, -1)
    return _tc_mlp(S, A, O, W1, b1r, W2, b2, block_m=2048)
